# baseline (speedup 1.0000x reference)
import functools

import jax
import jax.numpy as jnp
from jax import lax
from jax.experimental import pallas as pl
from jax.experimental.pallas import tpu as pltpu
from jax.experimental.pallas import tpu_sc as plsc

NUM_LABELS = 1000
LATENT_DIM = 512
TOPK = 10
BLOCK_B = 1024
BATCH = 16384

_NEG_INF = float("-inf")

NW = 32
RPW = BATCH // NW
CH = 8
NCH = RPW // CH
ROW_PAD = 1024


def _tc_logits_block(z_ref, e_ref, esq_ref, logits_ref, pad_ref):
    z = z_ref[...]
    e = e_ref[...]
    z_sq = jnp.sum(z * z, axis=1, keepdims=True)
    e_sq = esq_ref[...]
    dots = jax.lax.dot_general(
        z, e, (((1,), (1,)), ((), ())), preferred_element_type=jnp.float32)
    logits = 2.0 * dots - z_sq - e_sq
    logits_ref[...] = logits
    pad_ref[:, : NUM_LABELS] = logits
    pad_ref[:, NUM_LABELS:] = jnp.full(
        (logits.shape[0], ROW_PAD - NUM_LABELS), _NEG_INF, jnp.float32)


def _tc_logits(z, label_emb, e_sq):
    n_blocks = BATCH // BLOCK_B
    return pl.pallas_call(
        _tc_logits_block,
        grid=(n_blocks,),
        in_specs=[
            pl.BlockSpec((BLOCK_B, LATENT_DIM), lambda i: (i, 0)),
            pl.BlockSpec((NUM_LABELS, LATENT_DIM), lambda i: (0, 0)),
            pl.BlockSpec((1, NUM_LABELS), lambda i: (0, 0)),
        ],
        out_specs=(
            pl.BlockSpec((BLOCK_B, NUM_LABELS), lambda i: (i, 0)),
            pl.BlockSpec((BLOCK_B, ROW_PAD), lambda i: (i, 0)),
        ),
        out_shape=(
            jax.ShapeDtypeStruct((BATCH, NUM_LABELS), jnp.float32),
            jax.ShapeDtypeStruct((BATCH, ROW_PAD), jnp.float32),
        ),
    )(z, label_emb, e_sq)


@functools.partial(
    pl.kernel,
    mesh=plsc.VectorSubcoreMesh(core_axis_name="c", subcore_axis_name="s"),
    out_type=(
        jax.ShapeDtypeStruct((BATCH, 16), jnp.float32),
        jax.ShapeDtypeStruct((BATCH, 16), jnp.int32),
    ),
    scratch_types=[
        pltpu.VMEM((CH, ROW_PAD), jnp.float32),
        pltpu.VMEM((CH, 16), jnp.float32),
        pltpu.VMEM((CH, 16), jnp.int32),
    ],
)
def _sc_topk(pad_hbm, vals_hbm, idx_hbm, buf, vbuf, ibuf):
    wid = lax.axis_index("s") * 2 + lax.axis_index("c")
    base = wid * RPW
    iota = lax.iota(jnp.int32, 16)
    neg = jnp.full((16,), _NEG_INF, jnp.float32)
    bigl = jnp.full((16,), 1.0e9, jnp.float32)

    def _shuffle(x, idx):
        dnums = lax.GatherDimensionNumbers(
            offset_dims=(), collapsed_slice_dims=(0,), start_index_map=(0,))
        return lax.gather(
            x, idx[:, None], dnums, (1,),
            mode=lax.GatherScatterMode.PROMISE_IN_BOUNDS)

    def _xlane_max(x):
        for sh in (8, 4, 2, 1):
            x = jnp.maximum(x, _shuffle(x, iota ^ sh))
        return x

    def _xlane_min(x):
        for sh in (8, 4, 2, 1):
            x = jnp.minimum(x, _shuffle(x, iota ^ sh))
        return x

    def chunk_body(jc, _):
        r0 = base + jc * CH
        pltpu.sync_copy(pad_hbm.at[pl.ds(r0, CH)], buf)
        for r in range(CH):

            def sweep(c, carry):
                L, I = carry
                o = pl.multiple_of(c * 16, 16)
                v = buf[r, pl.ds(o, 16)]
                li = (o + iota).astype(jnp.float32)
                for j in range(TOPK):
                    swap = v > L[j]
                    nv = jnp.where(swap, L[j], v)
                    ni = jnp.where(swap, I[j], li)
                    L[j] = jnp.where(swap, v, L[j])
                    I[j] = jnp.where(swap, li, I[j])
                    v, li = nv, ni
                return (L, I)

            L0 = [neg] * TOPK
            I0 = [bigl] * TOPK
            L, I = lax.fori_loop(0, ROW_PAD // 16, sweep, (L0, I0))

            def extract(k, carry):
                L, I, vvec, ivec = carry
                gm = _xlane_max(L[0])
                hit = L[0] == gm
                gi = _xlane_min(jnp.where(hit, I[0], bigl))
                kk = iota == k
                vvec = jnp.where(kk, gm, vvec)
                ivec = jnp.where(kk, gi, ivec)
                lm = hit & (I[0] == gi)
                for j in range(TOPK - 1):
                    L[j] = jnp.where(lm, L[j + 1], L[j])
                    I[j] = jnp.where(lm, I[j + 1], I[j])
                L[TOPK - 1] = jnp.where(lm, neg, L[TOPK - 1])
                I[TOPK - 1] = jnp.where(lm, bigl, I[TOPK - 1])
                return (L, I, vvec, ivec)

            _, _, vvec, ivec = lax.fori_loop(
                0, TOPK, extract, (L, I, neg, bigl))
            vbuf[r, :] = vvec
            ibuf[r, :] = ivec.astype(jnp.int32)
        pltpu.sync_copy(vbuf, vals_hbm.at[pl.ds(r0, CH)])
        pltpu.sync_copy(ibuf, idx_hbm.at[pl.ds(r0, CH)])
        return 0

    lax.fori_loop(0, NCH, chunk_body, 0)


@jax.jit
def kernel(z, label_emb):
    e_sq = jnp.sum(label_emb * label_emb, axis=1)[None, :]
    logits, logits_pad = _tc_logits(z, label_emb, e_sq)
    vals16, idx16 = _sc_topk(logits_pad)
    return logits, vals16[:, :TOPK], idx16[:, :TOPK]


# speedup vs baseline: 2.4713x; 2.4713x over previous
import jax
import jax.numpy as jnp
from jax.experimental import pallas as pl
from jax.experimental.pallas import tpu as pltpu

NUM_LABELS = 1000
LATENT_DIM = 512
TOPK = 10
BLOCK_B = 2048

_NEG_INF = float("-inf")


def _lwal_block(z_ref, e_ref, esq_ref, logits_ref, vals_ref, idx_ref):
    z = z_ref[...]
    e = e_ref[...]
    z_sq = jnp.sum(z * z, axis=1, keepdims=True)
    e_sq = esq_ref[...]
    dots = jax.lax.dot_general(
        z, e, (((1,), (1,)), ((), ())), preferred_element_type=jnp.float32
    )
    logits = 2.0 * dots - z_sq - e_sq
    logits_ref[...] = logits

    bb = logits.shape[0]
    fiota = jax.lax.broadcasted_iota(
        jnp.int32, (bb, NUM_LABELS), 1).astype(jnp.float32)
    acc = logits
    for k in range(TOPK):
        m = jnp.max(acc, axis=1, keepdims=True)
        is_max = acc == m
        arg = jnp.min(jnp.where(is_max, fiota, 1024.0), axis=1,
                      keepdims=True)
        vals_ref[:, k] = m[:, 0]
        idx_ref[:, k] = arg[:, 0].astype(jnp.int32)
        acc = jnp.where(fiota == arg, _NEG_INF, acc)


@jax.jit
def kernel(z, label_emb):
    batch = z.shape[0]
    n_blocks = batch // BLOCK_B
    e_sq = jnp.sum(label_emb * label_emb, axis=1)[None, :]

    grid = (n_blocks,)
    out_shapes = (
        jax.ShapeDtypeStruct((batch, NUM_LABELS), jnp.float32),
        jax.ShapeDtypeStruct((batch, TOPK), jnp.float32),
        jax.ShapeDtypeStruct((batch, TOPK), jnp.int32),
    )
    logits, vals, idx = pl.pallas_call(
        _lwal_block,
        grid=grid,
        in_specs=[
            pl.BlockSpec((BLOCK_B, LATENT_DIM), lambda i: (i, 0)),
            pl.BlockSpec((NUM_LABELS, LATENT_DIM), lambda i: (0, 0)),
            pl.BlockSpec((1, NUM_LABELS), lambda i: (0, 0)),
        ],
        out_specs=(
            pl.BlockSpec((BLOCK_B, NUM_LABELS), lambda i: (i, 0)),
            pl.BlockSpec((BLOCK_B, TOPK), lambda i: (i, 0)),
            pl.BlockSpec((BLOCK_B, TOPK), lambda i: (i, 0)),
        ),
        out_shape=out_shapes,
    )(z, label_emb, e_sq)
    return logits, vals, idx


# speedup vs baseline: 2.4904x; 1.0077x over previous
import jax
import jax.numpy as jnp
from jax.experimental import pallas as pl
from jax.experimental.pallas import tpu as pltpu

NUM_LABELS = 1000
LATENT_DIM = 512
TOPK = 10
BLOCK_B = 1024

_NEG_INF = float("-inf")


def _lwal_block(z_ref, e_ref, esq_ref, logits_ref, vals_ref, idx_ref):
    z = z_ref[...]
    e = e_ref[...]
    z_sq = jnp.sum(z * z, axis=1, keepdims=True)
    e_sq = esq_ref[...]
    dots = jax.lax.dot_general(
        z, e, (((1,), (1,)), ((), ())), preferred_element_type=jnp.float32
    )
    logits = 2.0 * dots - z_sq - e_sq
    logits_ref[...] = logits

    bb = logits.shape[0]
    fiota = jax.lax.broadcasted_iota(
        jnp.int32, (bb, NUM_LABELS), 1).astype(jnp.float32)
    acc = logits
    for k in range(TOPK):
        m = jnp.max(acc, axis=1, keepdims=True)
        is_max = acc == m
        arg = jnp.min(jnp.where(is_max, fiota, 1024.0), axis=1,
                      keepdims=True)
        vals_ref[:, k] = m[:, 0]
        idx_ref[:, k] = arg[:, 0].astype(jnp.int32)
        acc = jnp.where(fiota == arg, _NEG_INF, acc)


@jax.jit
def kernel(z, label_emb):
    batch = z.shape[0]
    n_blocks = batch // BLOCK_B
    e_sq = jnp.sum(label_emb * label_emb, axis=1)[None, :]

    grid = (n_blocks,)
    out_shapes = (
        jax.ShapeDtypeStruct((batch, NUM_LABELS), jnp.float32),
        jax.ShapeDtypeStruct((batch, TOPK), jnp.float32),
        jax.ShapeDtypeStruct((batch, TOPK), jnp.int32),
    )
    logits, vals, idx = pl.pallas_call(
        _lwal_block,
        grid=grid,
        in_specs=[
            pl.BlockSpec((BLOCK_B, LATENT_DIM), lambda i: (i, 0)),
            pl.BlockSpec((NUM_LABELS, LATENT_DIM), lambda i: (0, 0)),
            pl.BlockSpec((1, NUM_LABELS), lambda i: (0, 0)),
        ],
        out_specs=(
            pl.BlockSpec((BLOCK_B, NUM_LABELS), lambda i: (i, 0)),
            pl.BlockSpec((BLOCK_B, TOPK), lambda i: (i, 0)),
            pl.BlockSpec((BLOCK_B, TOPK), lambda i: (i, 0)),
        ),
        out_shape=out_shapes,
    )(z, label_emb, e_sq)
    return logits, vals, idx
